# baseline (speedup 1.0000x reference)
import functools

import jax
import jax.numpy as jnp
from jax import lax
from jax.experimental import pallas as pl
from jax.experimental.pallas import tpu as pltpu
from jax.experimental.pallas import tpu_sc as plsc

N = 100000
E = 3200000
M = 1024
KE = 14.399645351950548
KEHALF = KE / 2.0
CUTOFF = 10.0

NC = 2
NS = 16
L = 16
NW = NC * NS
E_W = E // NW
CHUNK = 2048
NCH = E_W // CHUNK
REM = E_W - NCH * CHUNK

Q = 2047.0

_N_PAD = 100352

def _pack_body(prm_ref, z_ref, m_ref, out_ref):
    p = prm_ref[0]
    z = z_ref[...]
    zq = jnp.round(z * Q).astype(jnp.uint32)
    zpq = jnp.round(z ** p * Q).astype(jnp.uint32)
    mq = m_ref[...].astype(jnp.uint32)
    word = (mq << 22) | (zq << 11) | zpq
    out_ref[...] = word.astype(jnp.int32)


def _build_table(Z, idx_m, apow):
    zp = jnp.pad(Z, (0, _N_PAD - N)).reshape(_N_PAD // 128, 128)
    mp = jnp.pad(idx_m, (0, _N_PAD - N)).reshape(_N_PAD // 128, 128)
    prm = jax.nn.softplus(apow).reshape(1)
    out = pl.pallas_call(
        _pack_body,
        out_shape=jax.ShapeDtypeStruct(zp.shape, jnp.int32),
        in_specs=[
            pl.BlockSpec(memory_space=pltpu.SMEM),
            pl.BlockSpec(zp.shape, lambda: (0, 0)),
            pl.BlockSpec(mp.shape, lambda: (0, 0)),
        ],
        out_specs=pl.BlockSpec(zp.shape, lambda: (0, 0)),
    )(prm, zp, mp)
    return out.reshape(-1)


_SQB = 25600


def _geo_body(r_ref, o_ref):
    x = r_ref[0, :]
    y = r_ref[1, :]
    z = r_ref[2, :]
    s = x * x + y * y + z * z
    d = jnp.sqrt(s)
    ir = lax.rsqrt(s)
    xc = d * (1.0 / CUTOFF)
    fc = 1.0 + xc * xc * xc * (-10.0 + xc * (15.0 - 6.0 * xc))
    fc = jnp.where(d < CUTOFF, fc, 0.0)
    db = lax.bitcast_convert_type(d.astype(jnp.bfloat16), jnp.uint16).astype(jnp.uint32)
    gb = lax.bitcast_convert_type((fc * ir).astype(jnp.bfloat16), jnp.uint16).astype(jnp.uint32)
    word = ((db << 16) | gb).astype(jnp.int32)
    o_ref[...] = word.reshape(_SQB // 128, 128)


def _build_geo(rT):
    out = pl.pallas_call(
        _geo_body,
        grid=(E // _SQB,),
        out_shape=jax.ShapeDtypeStruct((E // 128, 128), jnp.int32),
        in_specs=[pl.BlockSpec((3, _SQB), lambda b: (0, b))],
        out_specs=pl.BlockSpec((_SQB // 128, 128), lambda b: (b, 0)),
    )(rT)
    return out.reshape(E)


def _reduce_body(x_ref, o_ref):
    o_ref[...] = jnp.sum(x_ref[...], axis=0, keepdims=True)


def _reduce_partials(partials):
    x = partials.reshape(NW * L, M)
    out = pl.pallas_call(
        _reduce_body,
        out_shape=jax.ShapeDtypeStruct((1, M), jnp.float32),
        in_specs=[pl.BlockSpec(x.shape, lambda: (0, 0))],
        out_specs=pl.BlockSpec((1, M), lambda: (0, 0)),
    )(x)
    return out.reshape(M)


def _sc_body(tbl, ii, jj, dgf, prm,
             out,
             tbl_v, ii_v, jj_v, dg_v,
             prm_v, acc, sem_t, sem_a):
    c = lax.axis_index("c")
    s = lax.axis_index("s")
    w = s * NC + c
    base = w * E_W

    pltpu.sync_copy(prm, prm_v)
    pv = prm_v[pl.ds(0, 16)]
    sak = [pv[k] for k in range(4)]
    cnk = [pv[4 + k] for k in range(4)]

    cpt = pltpu.async_copy(tbl, tbl_v, sem_t)

    zeros16 = jnp.zeros((L,), jnp.float32)

    def _zero(t, _):
        acc[pl.ds(t * L, L)] = zeros16
        return _

    lax.fori_loop(0, M, _zero, 0)

    lane = lax.iota(jnp.int32, L)
    laneoff = lane * M
    m11 = jnp.full((L,), 0x7FF, jnp.int32)

    cpt.wait()

    def _fire(eoff, n_edges, boff):
        pltpu.async_copy(ii.at[pl.ds(base + eoff, n_edges)],
                         ii_v.at[pl.ds(boff, n_edges)], sem_a)
        pltpu.async_copy(jj.at[pl.ds(base + eoff, n_edges)],
                         jj_v.at[pl.ds(boff, n_edges)], sem_a)
        pltpu.async_copy(dgf.at[pl.ds(base + eoff, n_edges)],
                         dg_v.at[pl.ds(boff, n_edges)], sem_a)

    def _wait(eoff, n_edges, boff):
        pltpu.make_async_copy(ii.at[pl.ds(base + eoff, n_edges)],
                              ii_v.at[pl.ds(boff, n_edges)], sem_a).wait()
        pltpu.make_async_copy(jj.at[pl.ds(base + eoff, n_edges)],
                              jj_v.at[pl.ds(boff, n_edges)], sem_a).wait()
        pltpu.make_async_copy(dgf.at[pl.ds(base + eoff, n_edges)],
                              dg_v.at[pl.ds(boff, n_edges)], sem_a).wait()

    def _edge16(off):
        dg = dg_v[pl.ds(off, L)]
        mhi = jnp.full((L,), -65536, jnp.int32)
        d = lax.bitcast_convert_type(dg & mhi, jnp.float32)
        g = lax.bitcast_convert_type(lax.shift_left(dg, 16), jnp.float32)

        iv = ii_v[pl.ds(off, L)]
        jv = jj_v[pl.ds(off, L)]
        wi = plsc.load_gather(tbl_v, [iv])
        wj = plsc.load_gather(tbl_v, [jv])

        mi = lax.shift_right_logical(wi, 22)
        Zi = (lax.shift_right_logical(wi, 11) & m11).astype(jnp.float32)
        Zj = (lax.shift_right_logical(wj, 11) & m11).astype(jnp.float32)
        zi = (wi & m11).astype(jnp.float32)
        zj = (wj & m11).astype(jnp.float32)

        t = (zi + zj) * d
        ssum = cnk[0] * jnp.exp(sak[0] * t)
        ssum = ssum + cnk[1] * jnp.exp(sak[1] * t)
        ssum = ssum + cnk[2] * jnp.exp(sak[2] * t)
        ssum = ssum + cnk[3] * jnp.exp(sak[3] * t)

        val = ssum * (Zi * Zj) * g
        plsc.addupdate_scatter(acc, [laneoff + mi], val)

    def _compute(n_vec, boff, unroll):
        def _vec(v, _):
            for u in range(unroll):
                _edge16(boff + (v * unroll + u) * L)
            return _

        lax.fori_loop(0, n_vec // unroll, _vec, 0)

    _fire(0, CHUNK, 0)

    def _chunk(ci, _):
        parity = (ci & 1) * CHUNK
        nparity = CHUNK - parity
        noff = lax.rem(ci + 1, NCH) * CHUNK
        _fire(noff, CHUNK, nparity)
        _wait(ci * CHUNK, CHUNK, parity)
        _compute(CHUNK // L, parity, 4)
        return _

    lax.fori_loop(0, NCH, _chunk, 0)
    _wait(0, CHUNK, (NCH & 1) * CHUNK)

    _fire(NCH * CHUNK, REM, 0)
    _wait(NCH * CHUNK, REM, 0)
    _compute(REM // L, 0, 2)

    pltpu.sync_copy(acc, out.at[pl.ds(w * L * M, L * M)])


@functools.partial(jax.jit, static_argnums=())
def kernel(Z, r_ij, idx_i, idx_j, idx_m, adiv, apow, a_vector, c_vector):
    tbl = _build_table(Z, idx_m, apow)

    spdiv = jax.nn.softplus(adiv)
    sak = jax.nn.softplus(a_vector)
    cc = jax.nn.softplus(c_vector)
    cn = cc / jnp.maximum(jnp.sum(jnp.abs(cc)), 1e-12)
    prm = jnp.concatenate([-sak * spdiv / Q,
                           (KEHALF / (Q * Q)) * cn,
                           jnp.zeros((8,), jnp.float32)])

    dgf = _build_geo(r_ij.T)

    mesh = plsc.VectorSubcoreMesh(core_axis_name="c", subcore_axis_name="s",
                                  num_cores=NC, num_subcores=NS)
    sc = pl.kernel(
        _sc_body,
        out_type=jax.ShapeDtypeStruct((NW * L * M,), jnp.float32),
        mesh=mesh,
        compiler_params=pltpu.CompilerParams(needs_layout_passes=False),
        scratch_types=[
            pltpu.VMEM((_N_PAD,), jnp.int32),
            pltpu.VMEM((2 * CHUNK,), jnp.int32),
            pltpu.VMEM((2 * CHUNK,), jnp.int32),
            pltpu.VMEM((2 * CHUNK,), jnp.int32),
            pltpu.VMEM((16,), jnp.float32),
            pltpu.VMEM((L * M,), jnp.float32),
            pltpu.SemaphoreType.DMA,
            pltpu.SemaphoreType.DMA,
        ],
    )
    partials = sc(tbl, idx_i, idx_j, dgf, prm)
    return _reduce_partials(partials)


# speedup vs baseline: 1.0133x; 1.0133x over previous
import functools

import jax
import jax.numpy as jnp
from jax import lax
from jax.experimental import pallas as pl
from jax.experimental.pallas import tpu as pltpu
from jax.experimental.pallas import tpu_sc as plsc

N = 100000
E = 3200000
M = 1024
KE = 14.399645351950548
KEHALF = KE / 2.0
CUTOFF = 10.0

NC = 2
NS = 16
L = 16
NW = NC * NS
E_W = E // NW
CHUNK = 1536
NCH = E_W // CHUNK
REM = E_W - NCH * CHUNK

Q = 2047.0

_N_PAD = 100352

def _pack_body(prm_ref, z_ref, m_ref, out_ref):
    p = prm_ref[0]
    z = z_ref[...]
    zq = jnp.round(z * Q).astype(jnp.uint32)
    zpq = jnp.round(z ** p * Q).astype(jnp.uint32)
    mq = m_ref[...].astype(jnp.uint32)
    word = (mq << 22) | (zq << 11) | zpq
    out_ref[...] = word.astype(jnp.int32)


def _build_table(Z, idx_m, apow):
    zp = jnp.pad(Z, (0, _N_PAD - N)).reshape(_N_PAD // 128, 128)
    mp = jnp.pad(idx_m, (0, _N_PAD - N)).reshape(_N_PAD // 128, 128)
    prm = jax.nn.softplus(apow).reshape(1)
    out = pl.pallas_call(
        _pack_body,
        out_shape=jax.ShapeDtypeStruct(zp.shape, jnp.int32),
        in_specs=[
            pl.BlockSpec(memory_space=pltpu.SMEM),
            pl.BlockSpec(zp.shape, lambda: (0, 0)),
            pl.BlockSpec(mp.shape, lambda: (0, 0)),
        ],
        out_specs=pl.BlockSpec(zp.shape, lambda: (0, 0)),
    )(prm, zp, mp)
    return out.reshape(-1)


_SQB = 25600


def _geo_body(r_ref, d_ref, g_ref):
    x = r_ref[0, :]
    y = r_ref[1, :]
    z = r_ref[2, :]
    s = x * x + y * y + z * z
    d = jnp.sqrt(s)
    ir = lax.rsqrt(s)
    xc = d * (1.0 / CUTOFF)
    fc = 1.0 + xc * xc * xc * (-10.0 + xc * (15.0 - 6.0 * xc))
    fc = jnp.where(d < CUTOFF, fc, 0.0)
    d_ref[...] = d.reshape(_SQB // 128, 128)
    g_ref[...] = (fc * ir).reshape(_SQB // 128, 128)


def _build_geo(rT):
    d_out, g_out = pl.pallas_call(
        _geo_body,
        grid=(E // _SQB,),
        out_shape=[jax.ShapeDtypeStruct((E // 128, 128), jnp.float32),
                   jax.ShapeDtypeStruct((E // 128, 128), jnp.float32)],
        in_specs=[pl.BlockSpec((3, _SQB), lambda b: (0, b))],
        out_specs=[pl.BlockSpec((_SQB // 128, 128), lambda b: (b, 0)),
                   pl.BlockSpec((_SQB // 128, 128), lambda b: (b, 0))],
    )(rT)
    return d_out.reshape(E), g_out.reshape(E)


def _reduce_body(x_ref, o_ref):
    o_ref[...] = jnp.sum(x_ref[...], axis=0, keepdims=True)


def _reduce_partials(partials):
    x = partials.reshape(NW * L, M)
    out = pl.pallas_call(
        _reduce_body,
        out_shape=jax.ShapeDtypeStruct((1, M), jnp.float32),
        in_specs=[pl.BlockSpec(x.shape, lambda: (0, 0))],
        out_specs=pl.BlockSpec((1, M), lambda: (0, 0)),
    )(x)
    return out.reshape(M)


def _sc_body(tbl, ii, jj, df, gf, prm,
             out,
             tbl_v, ii_v, jj_v, d_v, g_v,
             prm_v, acc, sem_t, sem_a):
    c = lax.axis_index("c")
    s = lax.axis_index("s")
    w = s * NC + c
    base = w * E_W

    pltpu.sync_copy(prm, prm_v)
    pv = prm_v[pl.ds(0, 16)]
    sak = [pv[k] for k in range(4)]
    cnk = [pv[4 + k] for k in range(4)]

    cpt = pltpu.async_copy(tbl, tbl_v, sem_t)

    zeros16 = jnp.zeros((L,), jnp.float32)

    def _zero(t, _):
        acc[pl.ds(t * L, L)] = zeros16
        return _

    lax.fori_loop(0, M, _zero, 0)

    lane = lax.iota(jnp.int32, L)
    laneoff = lane * M
    m11 = jnp.full((L,), 0x7FF, jnp.int32)

    cpt.wait()

    def _fire(eoff, n_edges, boff):
        pltpu.async_copy(ii.at[pl.ds(base + eoff, n_edges)],
                         ii_v.at[pl.ds(boff, n_edges)], sem_a)
        pltpu.async_copy(jj.at[pl.ds(base + eoff, n_edges)],
                         jj_v.at[pl.ds(boff, n_edges)], sem_a)
        pltpu.async_copy(df.at[pl.ds(base + eoff, n_edges)],
                         d_v.at[pl.ds(boff, n_edges)], sem_a)
        pltpu.async_copy(gf.at[pl.ds(base + eoff, n_edges)],
                         g_v.at[pl.ds(boff, n_edges)], sem_a)

    def _wait(eoff, n_edges, boff):
        pltpu.make_async_copy(ii.at[pl.ds(base + eoff, n_edges)],
                              ii_v.at[pl.ds(boff, n_edges)], sem_a).wait()
        pltpu.make_async_copy(jj.at[pl.ds(base + eoff, n_edges)],
                              jj_v.at[pl.ds(boff, n_edges)], sem_a).wait()
        pltpu.make_async_copy(df.at[pl.ds(base + eoff, n_edges)],
                              d_v.at[pl.ds(boff, n_edges)], sem_a).wait()
        pltpu.make_async_copy(gf.at[pl.ds(base + eoff, n_edges)],
                              g_v.at[pl.ds(boff, n_edges)], sem_a).wait()

    def _edge16(off):
        d = d_v[pl.ds(off, L)]
        g = g_v[pl.ds(off, L)]

        iv = ii_v[pl.ds(off, L)]
        jv = jj_v[pl.ds(off, L)]
        wi = plsc.load_gather(tbl_v, [iv])
        wj = plsc.load_gather(tbl_v, [jv])

        mi = lax.shift_right_logical(wi, 22)
        Zi = (lax.shift_right_logical(wi, 11) & m11).astype(jnp.float32)
        Zj = (lax.shift_right_logical(wj, 11) & m11).astype(jnp.float32)
        zi = (wi & m11).astype(jnp.float32)
        zj = (wj & m11).astype(jnp.float32)

        t = (zi + zj) * d
        ssum = cnk[0] * jnp.exp(sak[0] * t)
        ssum = ssum + cnk[1] * jnp.exp(sak[1] * t)
        ssum = ssum + cnk[2] * jnp.exp(sak[2] * t)
        ssum = ssum + cnk[3] * jnp.exp(sak[3] * t)

        val = ssum * (Zi * Zj) * g
        plsc.addupdate_scatter(acc, [laneoff + mi], val)

    def _compute(n_vec, boff, unroll):
        def _vec(v, _):
            for u in range(unroll):
                _edge16(boff + (v * unroll + u) * L)
            return _

        lax.fori_loop(0, n_vec // unroll, _vec, 0)

    _fire(0, CHUNK, 0)

    def _chunk(ci, _):
        parity = (ci & 1) * CHUNK
        nparity = CHUNK - parity
        noff = lax.rem(ci + 1, NCH) * CHUNK
        _fire(noff, CHUNK, nparity)
        _wait(ci * CHUNK, CHUNK, parity)
        _compute(CHUNK // L, parity, 8)
        return _

    lax.fori_loop(0, NCH, _chunk, 0)
    _wait(0, CHUNK, (NCH & 1) * CHUNK)

    _fire(NCH * CHUNK, REM, 0)
    _wait(NCH * CHUNK, REM, 0)
    _compute(REM // L, 0, 2)

    pltpu.sync_copy(acc, out.at[pl.ds(w * L * M, L * M)])


@functools.partial(jax.jit, static_argnums=())
def kernel(Z, r_ij, idx_i, idx_j, idx_m, adiv, apow, a_vector, c_vector):
    tbl = _build_table(Z, idx_m, apow)

    spdiv = jax.nn.softplus(adiv)
    sak = jax.nn.softplus(a_vector)
    cc = jax.nn.softplus(c_vector)
    cn = cc / jnp.maximum(jnp.sum(jnp.abs(cc)), 1e-12)
    prm = jnp.concatenate([-sak * spdiv / Q,
                           (KEHALF / (Q * Q)) * cn,
                           jnp.zeros((8,), jnp.float32)])

    df, gf = _build_geo(r_ij.T)

    mesh = plsc.VectorSubcoreMesh(core_axis_name="c", subcore_axis_name="s",
                                  num_cores=NC, num_subcores=NS)
    sc = pl.kernel(
        _sc_body,
        out_type=jax.ShapeDtypeStruct((NW * L * M,), jnp.float32),
        mesh=mesh,
        compiler_params=pltpu.CompilerParams(needs_layout_passes=False),
        scratch_types=[
            pltpu.VMEM((_N_PAD,), jnp.int32),
            pltpu.VMEM((2 * CHUNK,), jnp.int32),
            pltpu.VMEM((2 * CHUNK,), jnp.int32),
            pltpu.VMEM((2 * CHUNK,), jnp.float32),
            pltpu.VMEM((2 * CHUNK,), jnp.float32),
            pltpu.VMEM((16,), jnp.float32),
            pltpu.VMEM((L * M,), jnp.float32),
            pltpu.SemaphoreType.DMA,
            pltpu.SemaphoreType.DMA,
        ],
    )
    partials = sc(tbl, idx_i, idx_j, df, gf, prm)
    return _reduce_partials(partials)


# speedup vs baseline: 1.0166x; 1.0032x over previous
import functools

import jax
import jax.numpy as jnp
from jax import lax
from jax.experimental import pallas as pl
from jax.experimental.pallas import tpu as pltpu
from jax.experimental.pallas import tpu_sc as plsc

N = 100000
E = 3200000
M = 1024
KE = 14.399645351950548
KEHALF = KE / 2.0
CUTOFF = 10.0

NC = 2
NS = 16
L = 16
NW = NC * NS
E_W = E // NW
CHUNK = 1536
NCH = E_W // CHUNK
REM = E_W - NCH * CHUNK

Q = 2047.0

_N_PAD = 100352

def _pack_body(prm_ref, z_ref, m_ref, out_ref):
    p = prm_ref[0]
    z = z_ref[...]
    zq = jnp.round(z * Q).astype(jnp.uint32)
    zpq = jnp.round(z ** p * Q).astype(jnp.uint32)
    mq = m_ref[...].astype(jnp.uint32)
    word = (mq << 22) | (zq << 11) | zpq
    out_ref[...] = word.astype(jnp.int32)


def _build_table(Z, idx_m, apow):
    zp = jnp.pad(Z, (0, _N_PAD - N)).reshape(_N_PAD // 128, 128)
    mp = jnp.pad(idx_m, (0, _N_PAD - N)).reshape(_N_PAD // 128, 128)
    prm = jax.nn.softplus(apow).reshape(1)
    out = pl.pallas_call(
        _pack_body,
        out_shape=jax.ShapeDtypeStruct(zp.shape, jnp.int32),
        in_specs=[
            pl.BlockSpec(memory_space=pltpu.SMEM),
            pl.BlockSpec(zp.shape, lambda: (0, 0)),
            pl.BlockSpec(mp.shape, lambda: (0, 0)),
        ],
        out_specs=pl.BlockSpec(zp.shape, lambda: (0, 0)),
    )(prm, zp, mp)
    return out.reshape(-1)


_SQB = 25600


def _geo_body(prm_ref, r_ref, z_ref, m_ref, d_ref, g_ref, tbl_ref):
    @pl.when(pl.program_id(0) == 0)
    def _pack():
        p = prm_ref[0]
        zz = z_ref[...]
        zq = jnp.round(zz * Q).astype(jnp.uint32)
        zpq = jnp.round(zz ** p * Q).astype(jnp.uint32)
        mq = m_ref[...].astype(jnp.uint32)
        tbl_ref[...] = ((mq << 22) | (zq << 11) | zpq).astype(jnp.int32)

    x = r_ref[0, :]
    y = r_ref[1, :]
    z = r_ref[2, :]
    s = x * x + y * y + z * z
    d = jnp.sqrt(s)
    ir = lax.rsqrt(s)
    xc = d * (1.0 / CUTOFF)
    fc = 1.0 + xc * xc * xc * (-10.0 + xc * (15.0 - 6.0 * xc))
    fc = jnp.where(d < CUTOFF, fc, 0.0)
    d_ref[...] = d.reshape(_SQB // 128, 128)
    g_ref[...] = (fc * ir).reshape(_SQB // 128, 128)


def _build_geo(rT, Z, idx_m, apow):
    zp = jnp.pad(Z, (0, _N_PAD - N)).reshape(_N_PAD // 128, 128)
    mp = jnp.pad(idx_m, (0, _N_PAD - N)).reshape(_N_PAD // 128, 128)
    prm = jax.nn.softplus(apow).reshape(1)
    d_out, g_out, tbl = pl.pallas_call(
        _geo_body,
        grid=(E // _SQB,),
        out_shape=[jax.ShapeDtypeStruct((E // 128, 128), jnp.float32),
                   jax.ShapeDtypeStruct((E // 128, 128), jnp.float32),
                   jax.ShapeDtypeStruct(zp.shape, jnp.int32)],
        in_specs=[pl.BlockSpec(memory_space=pltpu.SMEM),
                  pl.BlockSpec((3, _SQB), lambda b: (0, b)),
                  pl.BlockSpec(zp.shape, lambda b: (0, 0)),
                  pl.BlockSpec(mp.shape, lambda b: (0, 0))],
        out_specs=[pl.BlockSpec((_SQB // 128, 128), lambda b: (b, 0)),
                   pl.BlockSpec((_SQB // 128, 128), lambda b: (b, 0)),
                   pl.BlockSpec(zp.shape, lambda b: (0, 0))],
    )(prm, rT, zp, mp)
    return d_out.reshape(E), g_out.reshape(E), tbl.reshape(-1)


def _reduce_body(x_ref, o_ref):
    o_ref[...] = jnp.sum(x_ref[...], axis=0, keepdims=True)


def _reduce_partials(partials):
    x = partials.reshape(NW * L, M)
    out = pl.pallas_call(
        _reduce_body,
        out_shape=jax.ShapeDtypeStruct((1, M), jnp.float32),
        in_specs=[pl.BlockSpec(x.shape, lambda: (0, 0))],
        out_specs=pl.BlockSpec((1, M), lambda: (0, 0)),
    )(x)
    return out.reshape(M)


def _sc_body(tbl, ii, jj, df, gf, prm,
             out,
             tbl_v, ii_v, jj_v, d_v, g_v,
             prm_v, acc, sem_t, sem_a):
    c = lax.axis_index("c")
    s = lax.axis_index("s")
    w = s * NC + c
    base = w * E_W

    pltpu.sync_copy(prm, prm_v)
    pv = prm_v[pl.ds(0, 16)]
    sak = [pv[k] for k in range(4)]
    cnk = [pv[4 + k] for k in range(4)]

    cpt = pltpu.async_copy(tbl, tbl_v, sem_t)

    zeros16 = jnp.zeros((L,), jnp.float32)

    def _zero(t, _):
        acc[pl.ds(t * L, L)] = zeros16
        return _

    lax.fori_loop(0, M, _zero, 0)

    lane = lax.iota(jnp.int32, L)
    laneoff = lane * M
    m11 = jnp.full((L,), 0x7FF, jnp.int32)

    cpt.wait()

    def _fire(eoff, n_edges, boff):
        pltpu.async_copy(ii.at[pl.ds(base + eoff, n_edges)],
                         ii_v.at[pl.ds(boff, n_edges)], sem_a)
        pltpu.async_copy(jj.at[pl.ds(base + eoff, n_edges)],
                         jj_v.at[pl.ds(boff, n_edges)], sem_a)
        pltpu.async_copy(df.at[pl.ds(base + eoff, n_edges)],
                         d_v.at[pl.ds(boff, n_edges)], sem_a)
        pltpu.async_copy(gf.at[pl.ds(base + eoff, n_edges)],
                         g_v.at[pl.ds(boff, n_edges)], sem_a)

    def _wait(eoff, n_edges, boff):
        pltpu.make_async_copy(ii.at[pl.ds(base + eoff, n_edges)],
                              ii_v.at[pl.ds(boff, n_edges)], sem_a).wait()
        pltpu.make_async_copy(jj.at[pl.ds(base + eoff, n_edges)],
                              jj_v.at[pl.ds(boff, n_edges)], sem_a).wait()
        pltpu.make_async_copy(df.at[pl.ds(base + eoff, n_edges)],
                              d_v.at[pl.ds(boff, n_edges)], sem_a).wait()
        pltpu.make_async_copy(gf.at[pl.ds(base + eoff, n_edges)],
                              g_v.at[pl.ds(boff, n_edges)], sem_a).wait()

    def _edge16(off):
        d = d_v[pl.ds(off, L)]
        g = g_v[pl.ds(off, L)]

        iv = ii_v[pl.ds(off, L)]
        jv = jj_v[pl.ds(off, L)]
        wi = plsc.load_gather(tbl_v, [iv])
        wj = plsc.load_gather(tbl_v, [jv])

        mi = lax.shift_right_logical(wi, 22)
        Zi = (lax.shift_right_logical(wi, 11) & m11).astype(jnp.float32)
        Zj = (lax.shift_right_logical(wj, 11) & m11).astype(jnp.float32)
        zi = (wi & m11).astype(jnp.float32)
        zj = (wj & m11).astype(jnp.float32)

        t = (zi + zj) * d
        ssum = cnk[0] * jnp.exp(sak[0] * t)
        ssum = ssum + cnk[1] * jnp.exp(sak[1] * t)
        ssum = ssum + cnk[2] * jnp.exp(sak[2] * t)
        ssum = ssum + cnk[3] * jnp.exp(sak[3] * t)

        val = ssum * (Zi * Zj) * g
        plsc.addupdate_scatter(acc, [laneoff + mi], val)

    def _compute(n_vec, boff, unroll):
        def _vec(v, _):
            for u in range(unroll):
                _edge16(boff + (v * unroll + u) * L)
            return _

        lax.fori_loop(0, n_vec // unroll, _vec, 0)

    _fire(0, CHUNK, 0)

    def _chunk(ci, _):
        parity = (ci & 1) * CHUNK
        nparity = CHUNK - parity
        noff = lax.rem(ci + 1, NCH) * CHUNK
        _fire(noff, CHUNK, nparity)
        _wait(ci * CHUNK, CHUNK, parity)
        _compute(CHUNK // L, parity, 8)
        return _

    lax.fori_loop(0, NCH, _chunk, 0)
    _wait(0, CHUNK, (NCH & 1) * CHUNK)

    _fire(NCH * CHUNK, REM, 0)
    _wait(NCH * CHUNK, REM, 0)
    _compute(REM // L, 0, 2)

    pltpu.sync_copy(acc, out.at[pl.ds(w * L * M, L * M)])


@functools.partial(jax.jit, static_argnums=())
def kernel(Z, r_ij, idx_i, idx_j, idx_m, adiv, apow, a_vector, c_vector):
    spdiv = jax.nn.softplus(adiv)
    sak = jax.nn.softplus(a_vector)
    cc = jax.nn.softplus(c_vector)
    cn = cc / jnp.maximum(jnp.sum(jnp.abs(cc)), 1e-12)
    prm = jnp.concatenate([-sak * spdiv / Q,
                           (KEHALF / (Q * Q)) * cn,
                           jnp.zeros((8,), jnp.float32)])

    df, gf, tbl = _build_geo(r_ij.T, Z, idx_m, apow)

    mesh = plsc.VectorSubcoreMesh(core_axis_name="c", subcore_axis_name="s",
                                  num_cores=NC, num_subcores=NS)
    sc = pl.kernel(
        _sc_body,
        out_type=jax.ShapeDtypeStruct((NW * L * M,), jnp.float32),
        mesh=mesh,
        compiler_params=pltpu.CompilerParams(needs_layout_passes=False),
        scratch_types=[
            pltpu.VMEM((_N_PAD,), jnp.int32),
            pltpu.VMEM((2 * CHUNK,), jnp.int32),
            pltpu.VMEM((2 * CHUNK,), jnp.int32),
            pltpu.VMEM((2 * CHUNK,), jnp.float32),
            pltpu.VMEM((2 * CHUNK,), jnp.float32),
            pltpu.VMEM((16,), jnp.float32),
            pltpu.VMEM((L * M,), jnp.float32),
            pltpu.SemaphoreType.DMA,
            pltpu.SemaphoreType.DMA,
        ],
    )
    partials = sc(tbl, idx_i, idx_j, df, gf, prm)
    return _reduce_partials(partials)
